# baseline (speedup 1.0000x reference)
import functools

import jax
import jax.numpy as jnp
from jax import lax
from jax.experimental import pallas as pl
from jax.experimental.pallas import tpu as pltpu
from jax.experimental.pallas import tpu_sc as plsc

_NUM_CORES = 2
_NUM_SUBCORES = 16
_NUM_WORKERS = _NUM_CORES * _NUM_SUBCORES
_SPLITS = 4


def _make_copy(seq_len: int, d: int, dtype):
    rows_per_w = seq_len // _NUM_WORKERS
    rows_per_dma = rows_per_w // _SPLITS
    mesh = plsc.VectorSubcoreMesh(
        core_axis_name="c",
        subcore_axis_name="s",
        num_cores=_NUM_CORES,
        num_subcores=_NUM_SUBCORES,
    )

    @functools.partial(
        pl.kernel,
        out_type=jax.ShapeDtypeStruct((seq_len, d), dtype),
        mesh=mesh,
        scratch_types=[pltpu.SemaphoreType.DMA] * _SPLITS,
    )
    def copy_rows(table_hbm, out_hbm, *sems):
        wid = lax.axis_index("s") * _NUM_CORES + lax.axis_index("c")
        base = wid * rows_per_w
        copies = []
        for j in range(_SPLITS):
            off = base + j * rows_per_dma
            cp = pltpu.make_async_copy(
                table_hbm.at[pl.ds(off, rows_per_dma)],
                out_hbm.at[pl.ds(off, rows_per_dma)],
                sems[j],
            )
            cp.start()
            copies.append(cp)
        for cp in copies:
            cp.wait()

    return copy_rows


def kernel(x, pos_emb):
    seq_len = x.shape[1]
    d = pos_emb.shape[1]
    return _make_copy(seq_len, d, pos_emb.dtype)(pos_emb)


# speedup vs baseline: 16.1285x; 16.1285x over previous
import functools

import jax
import jax.numpy as jnp
from jax import lax
from jax.experimental import pallas as pl
from jax.experimental.pallas import tpu as pltpu
from jax.experimental.pallas import tpu_sc as plsc

_NUM_CORES = 2
_NUM_SUBCORES = 16
_NUM_WORKERS = _NUM_CORES * _NUM_SUBCORES
_CHUNK_ROWS = 32


def _make_copy(seq_len: int, d: int, dtype):
    rows_per_w = seq_len // _NUM_WORKERS
    n_chunks = rows_per_w // _CHUNK_ROWS
    mesh = plsc.VectorSubcoreMesh(
        core_axis_name="c",
        subcore_axis_name="s",
        num_cores=_NUM_CORES,
        num_subcores=_NUM_SUBCORES,
    )

    @functools.partial(
        pl.kernel,
        out_type=jax.ShapeDtypeStruct((seq_len, d), dtype),
        mesh=mesh,
        scratch_types=[
            pltpu.VMEM((_CHUNK_ROWS, d), dtype),
            pltpu.VMEM((_CHUNK_ROWS, d), dtype),
            pltpu.SemaphoreType.DMA,
            pltpu.SemaphoreType.DMA,
            pltpu.SemaphoreType.DMA,
            pltpu.SemaphoreType.DMA,
        ],
    )
    def copy_rows(table_hbm, out_hbm, buf0, buf1, gs0, gs1, ss0, ss1):
        wid = lax.axis_index("s") * _NUM_CORES + lax.axis_index("c")
        base = wid * rows_per_w
        bufs = (buf0, buf1)
        gsems = (gs0, gs1)
        ssems = (ss0, ss1)
        scatters = [None, None]
        for j in range(n_chunks):
            b = j & 1
            off = base + j * _CHUNK_ROWS
            if scatters[b] is not None:
                scatters[b].wait()
            gather = pltpu.make_async_copy(
                table_hbm.at[pl.ds(off, _CHUNK_ROWS)], bufs[b], gsems[b]
            )
            gather.start()
            gather.wait()
            scatter = pltpu.make_async_copy(
                bufs[b], out_hbm.at[pl.ds(off, _CHUNK_ROWS)], ssems[b]
            )
            scatter.start()
            scatters[b] = scatter
        for sc in scatters:
            if sc is not None:
                sc.wait()

    return copy_rows


def kernel(x, pos_emb):
    seq_len = x.shape[1]
    d = pos_emb.shape[1]
    return _make_copy(seq_len, d, pos_emb.dtype)(pos_emb)


# speedup vs baseline: 16.8372x; 1.0439x over previous
import functools

import jax
import jax.numpy as jnp
from jax import lax
from jax.experimental import pallas as pl
from jax.experimental.pallas import tpu as pltpu
from jax.experimental.pallas import tpu_sc as plsc

_NUM_CORES = 2
_NUM_SUBCORES = 16
_NUM_WORKERS = _NUM_CORES * _NUM_SUBCORES
_CHUNK_ROWS = 16
_NUM_BUFS = 7


def _make_copy(seq_len: int, d: int, dtype):
    rows_per_w = seq_len // _NUM_WORKERS
    n_chunks = rows_per_w // _CHUNK_ROWS
    n_bufs = min(_NUM_BUFS, n_chunks)
    mesh = plsc.VectorSubcoreMesh(
        core_axis_name="c",
        subcore_axis_name="s",
        num_cores=_NUM_CORES,
        num_subcores=_NUM_SUBCORES,
    )

    @functools.partial(
        pl.kernel,
        out_type=jax.ShapeDtypeStruct((seq_len, d), dtype),
        mesh=mesh,
        scratch_types=(
            [pltpu.VMEM((_CHUNK_ROWS, d), dtype)] * n_bufs
            + [pltpu.SemaphoreType.DMA] * (2 * n_bufs)
        ),
    )
    def copy_rows(table_hbm, out_hbm, *rest):
        bufs = rest[:n_bufs]
        gsems = rest[n_bufs : 2 * n_bufs]
        ssems = rest[2 * n_bufs :]
        wid = lax.axis_index("s") * _NUM_CORES + lax.axis_index("c")
        base = wid * rows_per_w

        def row_slice(ref, j):
            return ref.at[pl.ds(base + j * _CHUNK_ROWS, _CHUNK_ROWS)]

        gathers = []
        for j in range(n_bufs):
            g = pltpu.make_async_copy(row_slice(table_hbm, j), bufs[j], gsems[j])
            g.start()
            gathers.append(g)
        scatters = []
        for j in range(n_chunks):
            b = j % n_bufs
            if j >= n_bufs:
                scatters[j - n_bufs].wait()
                g = pltpu.make_async_copy(row_slice(table_hbm, j), bufs[b], gsems[b])
                g.start()
                gathers.append(g)
            gathers[j].wait()
            sc = pltpu.make_async_copy(bufs[b], row_slice(out_hbm, j), ssems[b])
            sc.start()
            scatters.append(sc)
        for j in range(max(0, n_chunks - n_bufs), n_chunks):
            scatters[j].wait()

    return copy_rows


def kernel(x, pos_emb):
    seq_len = x.shape[1]
    d = pos_emb.shape[1]
    return _make_copy(seq_len, d, pos_emb.dtype)(pos_emb)
